# Initial kernel scaffold; baseline (speedup 1.0000x reference)
import jax
import jax.numpy as jnp
from jax.experimental import pallas as pl


def kernel(x, edge_index, batch, ptr, text_output, conv1_W, conv1_b, conv2_W, conv2_b, conv3_W, conv3_b, mol_h1_W, mol_h1_b, mol_h2_W, mol_h2_b, text_h1_W, text_h1_b, text_h2_W, text_h2_b, ln1_w, ln1_b, ln2_w, ln2_b, temp):
    raise NotImplementedError("write your pallas kernel here")


# speedup vs baseline: 7.7826x; 7.7826x over previous
import functools

import jax
import jax.numpy as jnp
from jax import lax
from jax.experimental import pallas as pl
from jax.experimental.pallas import tpu as pltpu
from jax.experimental.pallas import tpu_sc as plsc

N = 10000
E = 160000
G = 64
HH = 128
K = 128
NBATCH = E // K
NSUB = 16
N_PAD = 10240
NPS = N_PAD // NSUB

_mesh = functools.partial(
    plsc.VectorSubcoreMesh, core_axis_name="c", subcore_axis_name="s")


def _sc_hist(dst3d, zeros_hbm, ones_hbm):
  @functools.partial(
      pl.kernel,
      out_type=jax.ShapeDtypeStruct((2 * N_PAD, 128), jnp.float32),
      mesh=_mesh(),
      scratch_types=[
          pltpu.VMEM((1, K), jnp.int32),
          pltpu.VMEM((K, 128), jnp.float32),
          pltpu.VMEM_SHARED((N_PAD, 128), jnp.float32),
      ],
  )
  def hist_kernel(dst_hbm, z_hbm, o_hbm, out, idx_v, ones_v, acc):
    c = lax.axis_index("c")
    s = lax.axis_index("s")
    w = c * NSUB + s

    pltpu.sync_copy(o_hbm, ones_v)

    @pl.loop(0, 5)
    def _(j):
      pltpu.sync_copy(z_hbm, acc.at[pl.ds(s * NPS + j * K, K)])

    plsc.subcore_barrier()

    @pl.loop(0, NBATCH // 32)
    def _(j):
      pltpu.sync_copy(dst_hbm.at[w + 32 * j], idx_v)
      pltpu.sync_copy(ones_v, acc.at[idx_v.at[0]], add=True)

    @pl.when(w < NBATCH - 32 * (NBATCH // 32))
    def _():
      pltpu.sync_copy(dst_hbm.at[w + 32 * (NBATCH // 32)], idx_v)
      pltpu.sync_copy(ones_v, acc.at[idx_v.at[0]], add=True)

    plsc.subcore_barrier()

    pltpu.sync_copy(acc.at[pl.ds(s * NPS, NPS)],
                    out.at[pl.ds(c * N_PAD + s * NPS, NPS)])

  return hist_kernel(dst3d, zeros_hbm, ones_hbm)


def _sc_agg(ys_all, src_both, dst3d, zeros_hbm):
  @functools.partial(
      pl.kernel,
      out_type=jax.ShapeDtypeStruct((2 * N_PAD, HH), jnp.float32),
      mesh=_mesh(),
      scratch_types=[
          pltpu.VMEM((1, K), jnp.int32),
          pltpu.VMEM((1, K), jnp.int32),
          pltpu.VMEM((K, HH), jnp.float32),
          pltpu.VMEM_SHARED((N_PAD, HH), jnp.float32),
      ],
  )
  def agg_kernel(hh_hbm, src_hbm, dst_hbm, z_hbm, out,
                 idxs_v, idxd_v, rows_v, acc):
    c = lax.axis_index("c")
    s = lax.axis_index("s")

    @pl.loop(0, 5)
    def _(j):
      pltpu.sync_copy(z_hbm, acc.at[pl.ds(s * NPS + j * K, K)])

    plsc.subcore_barrier()

    def do_batch(b):
      pltpu.sync_copy(src_hbm.at[c * NBATCH + b], idxs_v)
      pltpu.sync_copy(dst_hbm.at[b], idxd_v)
      pltpu.sync_copy(hh_hbm.at[idxs_v.at[0]], rows_v)
      pltpu.sync_copy(rows_v, acc.at[idxd_v.at[0]], add=True)

    @pl.loop(0, NBATCH // NSUB)
    def _(j):
      do_batch(s + NSUB * j)

    @pl.when(s < NBATCH - NSUB * (NBATCH // NSUB))
    def _():
      do_batch(s + NSUB * (NBATCH // NSUB))

    plsc.subcore_barrier()

    pltpu.sync_copy(acc.at[pl.ds(s * NPS, NPS)],
                    out.at[pl.ds(c * N_PAD + s * NPS, NPS)])

  return agg_kernel(ys_all, src_both, dst3d, zeros_hbm)


BLK = 1000


def _dinv_of(h0, h1):
  return lax.rsqrt(h0[:, 0:1] + h1[:, 0:1] + 1.0)


def _tc_layer1(x, W, hist0, hist1):
  def body(x_ref, w_ref, h0_ref, h1_ref, o_ref):
    dinv = _dinv_of(h0_ref[...], h1_ref[...])
    y = jnp.dot(x_ref[...], w_ref[...], preferred_element_type=jnp.float32,
                precision=lax.Precision.HIGHEST)
    o_ref[...] = y * dinv

  return pl.pallas_call(
      body,
      grid=(2 * (N // BLK),),
      in_specs=[
          pl.BlockSpec((BLK, 256), lambda i: (i % (N // BLK), 0)),
          pl.BlockSpec((256, HH), lambda i: (0, i // (N // BLK))),
          pl.BlockSpec((BLK, 128), lambda i: (i % (N // BLK), 0)),
          pl.BlockSpec((BLK, 128), lambda i: (i % (N // BLK), 0)),
      ],
      out_specs=pl.BlockSpec((BLK, HH), lambda i: (i, 0)),
      out_shape=jax.ShapeDtypeStruct((2 * N, HH), jnp.float32),
  )(x, W, hist0, hist1)


def _tc_layer(a0, a1, ys_all, hist0, hist1, b_prev, W):
  def body(a0_r, a1_r, p0_r, p1_r, h0_r, h1_r, b_r, w_r, o_ref):
    dinv = _dinv_of(h0_r[...], h1_r[...])
    a = jnp.concatenate([a0_r[...], a1_r[...]], axis=1)
    p = jnp.concatenate([p0_r[...], p1_r[...]], axis=1)
    h = jnp.maximum(dinv * (a + p) + b_r[...], 0.0)
    y = jnp.dot(h, w_r[...], preferred_element_type=jnp.float32,
                precision=lax.Precision.HIGHEST)
    o_ref[...] = y * dinv

  nb = N // BLK
  return pl.pallas_call(
      body,
      grid=(2 * nb,),
      in_specs=[
          pl.BlockSpec((BLK, HH), lambda i: (i % nb, 0)),
          pl.BlockSpec((BLK, HH), lambda i: (i % nb, 0)),
          pl.BlockSpec((BLK, HH), lambda i: (i % nb, 0)),
          pl.BlockSpec((BLK, HH), lambda i: (nb + i % nb, 0)),
          pl.BlockSpec((BLK, 128), lambda i: (i % nb, 0)),
          pl.BlockSpec((BLK, 128), lambda i: (i % nb, 0)),
          pl.BlockSpec((1, 256), lambda i: (0, 0)),
          pl.BlockSpec((256, HH), lambda i: (0, i // nb)),
      ],
      out_specs=pl.BlockSpec((BLK, HH), lambda i: (i, 0)),
      out_shape=jax.ShapeDtypeStruct((2 * N, HH), jnp.float32),
  )(a0, a1, ys_all, ys_all, hist0, hist1, b_prev, W)


def _tc_pool(a0, a1, ys_all, hist0, hist1, b3, batch2d):
  def body(a0_r, a1_r, p0_r, p1_r, h0_r, h1_r, b_r, bat_r, ps_ref, cnt_ref):
    i = pl.program_id(0)
    dinv = _dinv_of(h0_r[...], h1_r[...])
    a = jnp.concatenate([a0_r[...], a1_r[...]], axis=1)
    p = jnp.concatenate([p0_r[...], p1_r[...]], axis=1)
    mol = dinv * (a + p) + b_r[...]
    gids = lax.broadcasted_iota(jnp.int32, (G, BLK), 0)
    S = (bat_r[...][0] == gids)
    part = jnp.dot(S.astype(jnp.float32), mol,
                   preferred_element_type=jnp.float32,
                   precision=lax.Precision.HIGHEST)
    c_part = jnp.broadcast_to(
        jnp.sum(S.astype(jnp.float32), axis=1, keepdims=True), (G, 128))

    @pl.when(i == 0)
    def _():
      ps_ref[...] = part
      cnt_ref[...] = c_part

    @pl.when(i > 0)
    def _():
      ps_ref[...] += part
      cnt_ref[...] += c_part

  return pl.pallas_call(
      body,
      grid=(N // BLK,),
      in_specs=[
          pl.BlockSpec((BLK, HH), lambda i: (i, 0)),
          pl.BlockSpec((BLK, HH), lambda i: (i, 0)),
          pl.BlockSpec((BLK, HH), lambda i: (i, 0)),
          pl.BlockSpec((BLK, HH), lambda i: (N // BLK + i, 0)),
          pl.BlockSpec((BLK, 128), lambda i: (i, 0)),
          pl.BlockSpec((BLK, 128), lambda i: (i, 0)),
          pl.BlockSpec((1, 256), lambda i: (0, 0)),
          pl.BlockSpec((1, 1, BLK), lambda i: (i, 0, 0)),
      ],
      out_specs=[
          pl.BlockSpec((G, 256), lambda i: (0, 0)),
          pl.BlockSpec((G, 128), lambda i: (0, 0)),
      ],
      out_shape=[
          jax.ShapeDtypeStruct((G, 256), jnp.float32),
          jax.ShapeDtypeStruct((G, 128), jnp.float32),
      ],
  )(a0, a1, ys_all, ys_all, hist0, hist1, b3, batch2d)


def _tc_head(ps, cnt, text, mw1, mb1, mw2, mb2, tw1, tb1, tw2, tb2,
             l1w, l1b, l2w, l2b, temp11):
  def _ln(v, w, b):
    mu = jnp.mean(v, axis=-1, keepdims=True)
    var = jnp.mean(jnp.square(v - mu), axis=-1, keepdims=True)
    return (v - mu) * lax.rsqrt(var + 1e-5) * w + b

  def body(ps_r, cnt_r, t_r, mw1_r, mb1_r, mw2_r, mb2_r, tw1_r, tb1_r,
           tw2_r, tb2_r, l1w_r, l1b_r, l2w_r, l2b_r, tmp_r,
           tx_ref, xg_ref):
    cntv = jnp.maximum(cnt_r[:, 0:1], 1.0)
    pooled = ps_r[...] / cntv
    xg = jnp.maximum(
        jnp.dot(pooled, mw1_r[...], preferred_element_type=jnp.float32,
                precision=lax.Precision.HIGHEST)
        + mb1_r[...], 0.0)
    xg = jnp.dot(xg, mw2_r[...], preferred_element_type=jnp.float32,
                precision=lax.Precision.HIGHEST) + mb2_r[...]
    tx = jnp.tanh(
        jnp.dot(t_r[...], tw1_r[...], preferred_element_type=jnp.float32,
                precision=lax.Precision.HIGHEST)
        + tb1_r[...])
    tx = jnp.dot(tx, tw2_r[...], preferred_element_type=jnp.float32,
                precision=lax.Precision.HIGHEST) + tb2_r[...]
    sc = jnp.exp(tmp_r[0, 0])
    xg_ref[...] = _ln(xg, l1w_r[...], l1b_r[...]) * sc
    tx_ref[...] = _ln(tx, l2w_r[...], l2b_r[...]) * sc

  return pl.pallas_call(
      body,
      out_shape=[
          jax.ShapeDtypeStruct((G, 256), jnp.float32),
          jax.ShapeDtypeStruct((G, 256), jnp.float32),
      ],
  )(ps, cnt, text, mw1, mb1, mw2, mb2, tw1, tb1, tw2, tb2,
    l1w, l1b, l2w, l2b, temp11)


def kernel(x, edge_index, batch, ptr, text_output, conv1_W, conv1_b,
           conv2_W, conv2_b, conv3_W, conv3_b, mol_h1_W, mol_h1_b,
           mol_h2_W, mol_h2_b, text_h1_W, text_h1_b, text_h2_W, text_h2_b,
           ln1_w, ln1_b, ln2_w, ln2_b, temp):
  src_both = jnp.concatenate(
      [edge_index[0], edge_index[0] + N]).reshape(2 * NBATCH, 1, K)
  dst3d = edge_index[1].reshape(NBATCH, 1, K)
  zeros_c = jnp.zeros((K, 128), jnp.float32)
  ones_c = jnp.ones((K, 128), jnp.float32)

  hist = _sc_hist(dst3d, zeros_c, ones_c)
  hist0, hist1 = hist[:N_PAD], hist[N_PAD:]
  ys = _tc_layer1(x, conv1_W, hist0, hist1)
  agg = _sc_agg(ys, src_both, dst3d, zeros_c)
  a0, a1 = agg[:N_PAD], agg[N_PAD:]
  ys = _tc_layer(a0, a1, ys, hist0, hist1, conv1_b.reshape(1, 256), conv2_W)
  agg = _sc_agg(ys, src_both, dst3d, zeros_c)
  a0, a1 = agg[:N_PAD], agg[N_PAD:]
  ys = _tc_layer(a0, a1, ys, hist0, hist1, conv2_b.reshape(1, 256), conv3_W)
  agg = _sc_agg(ys, src_both, dst3d, zeros_c)
  a0, a1 = agg[:N_PAD], agg[N_PAD:]
  ps, cnt = _tc_pool(a0, a1, ys, hist0, hist1,
                     conv3_b.reshape(1, 256), batch.reshape(N // BLK, 1, BLK))
  tx, xg = _tc_head(
      ps, cnt, text_output[0], mol_h1_W, mol_h1_b.reshape(1, 512),
      mol_h2_W, mol_h2_b.reshape(1, 256), text_h1_W, text_h1_b.reshape(1, 512),
      text_h2_W, text_h2_b.reshape(1, 256), ln1_w.reshape(1, 256),
      ln1_b.reshape(1, 256), ln2_w.reshape(1, 256), ln2_b.reshape(1, 256),
      temp.reshape(1, 1))
  return (tx, xg)


# speedup vs baseline: 10.8668x; 1.3963x over previous
import functools

import jax
import jax.numpy as jnp
from jax import lax
from jax.experimental import pallas as pl
from jax.experimental.pallas import tpu as pltpu
from jax.experimental.pallas import tpu_sc as plsc

N = 10000
E = 160000
G = 64
HH = 128
K = 128
NBATCH = E // K
NSUB = 16
N_PAD = 10240
NPS = N_PAD // NSUB

_mesh = functools.partial(
    plsc.VectorSubcoreMesh, core_axis_name="c", subcore_axis_name="s")


def _sc_hist(dst3d, zeros_hbm, ones_hbm):
  @functools.partial(
      pl.kernel,
      out_type=jax.ShapeDtypeStruct((2 * N_PAD, 128), jnp.float32),
      mesh=_mesh(),
      scratch_types=[
          pltpu.VMEM((1, K), jnp.int32),
          pltpu.VMEM((K, 128), jnp.float32),
          pltpu.VMEM_SHARED((N_PAD, 128), jnp.float32),
      ],
  )
  def hist_kernel(dst_hbm, z_hbm, o_hbm, out, idx_v, ones_v, acc):
    c = lax.axis_index("c")
    s = lax.axis_index("s")
    w = c * NSUB + s

    pltpu.sync_copy(o_hbm, ones_v)

    @pl.loop(0, 5)
    def _(j):
      pltpu.sync_copy(z_hbm, acc.at[pl.ds(s * NPS + j * K, K)])

    plsc.subcore_barrier()

    @pl.loop(0, NBATCH // 32)
    def _(j):
      pltpu.sync_copy(dst_hbm.at[w + 32 * j], idx_v)
      pltpu.sync_copy(ones_v, acc.at[idx_v.at[0]], add=True)

    @pl.when(w < NBATCH - 32 * (NBATCH // 32))
    def _():
      pltpu.sync_copy(dst_hbm.at[w + 32 * (NBATCH // 32)], idx_v)
      pltpu.sync_copy(ones_v, acc.at[idx_v.at[0]], add=True)

    plsc.subcore_barrier()

    pltpu.sync_copy(acc.at[pl.ds(s * NPS, NPS)],
                    out.at[pl.ds(c * N_PAD + s * NPS, NPS)])

  return hist_kernel(dst3d, zeros_hbm, ones_hbm)


def _sc_agg(ys_all, src_both, dst3d, zeros_hbm):
  NJOB = NBATCH // NSUB

  @functools.partial(
      pl.kernel,
      out_type=jax.ShapeDtypeStruct((2 * N_PAD, HH), jnp.float32),
      mesh=_mesh(),
      scratch_types=[
          pltpu.VMEM((1, K), jnp.int32),
          pltpu.VMEM((1, K), jnp.int32),
          pltpu.VMEM((1, K), jnp.int32),
          pltpu.VMEM((1, K), jnp.int32),
          pltpu.VMEM((K, HH), jnp.float32),
          pltpu.VMEM((K, HH), jnp.float32),
          pltpu.SemaphoreType.DMA,
          pltpu.VMEM_SHARED((N_PAD, HH), jnp.float32),
      ],
  )
  def agg_kernel(hh_hbm, src_hbm, dst_hbm, z_hbm, out,
                 idxs0, idxs1, idxd0, idxd1, rows0, rows1, gsem, acc):
    c = lax.axis_index("c")
    s = lax.axis_index("s")
    idxs = (idxs0, idxs1)
    idxd = (idxd0, idxd1)
    rows = (rows0, rows1)

    @pl.loop(0, 5)
    def _(j):
      pltpu.sync_copy(z_hbm, acc.at[pl.ds(s * NPS + j * K, K)])

    def batch_of(j):
      return s + NSUB * j

    plsc.subcore_barrier()

    for p in range(2):
      pltpu.sync_copy(src_hbm.at[c * NBATCH + batch_of(p)], idxs[p])
      pltpu.sync_copy(dst_hbm.at[batch_of(p)], idxd[p])
      pltpu.async_copy(hh_hbm.at[idxs[p].at[0]], rows[p], gsem)

    @pl.loop(0, NJOB // 2)
    def _(j2):
      for p in range(2):
        jj = 2 * j2 + p
        pltpu.make_async_copy(hh_hbm.at[idxs[p].at[0]], rows[p], gsem).wait()
        pltpu.sync_copy(rows[p], acc.at[idxd[p].at[0]], add=True)

        @pl.when(jj + 2 < NJOB)
        def _():
          pltpu.sync_copy(src_hbm.at[c * NBATCH + batch_of(jj + 2)], idxs[p])
          pltpu.sync_copy(dst_hbm.at[batch_of(jj + 2)], idxd[p])
          pltpu.async_copy(hh_hbm.at[idxs[p].at[0]], rows[p], gsem)

    @pl.when(s < NBATCH - NSUB * NJOB)
    def _():
      b = s + NSUB * NJOB
      pltpu.sync_copy(src_hbm.at[c * NBATCH + b], idxs0)
      pltpu.sync_copy(dst_hbm.at[b], idxd0)
      pltpu.sync_copy(hh_hbm.at[idxs0.at[0]], rows0)
      pltpu.sync_copy(rows0, acc.at[idxd0.at[0]], add=True)

    plsc.subcore_barrier()

    pltpu.sync_copy(acc.at[pl.ds(s * NPS, NPS)],
                    out.at[pl.ds(c * N_PAD + s * NPS, NPS)])

  return agg_kernel(ys_all, src_both, dst3d, zeros_hbm)


BLK = 1000


def _dinv_of(h0, h1):
  return lax.rsqrt(h0[:, 0:1] + h1[:, 0:1] + 1.0)


def _tc_layer1(x, W, hist0, hist1):
  def body(x_ref, w_ref, h0_ref, h1_ref, o_ref):
    dinv = _dinv_of(h0_ref[...], h1_ref[...])
    y = jnp.dot(x_ref[...], w_ref[...], preferred_element_type=jnp.float32,
                precision=lax.Precision.HIGHEST)
    o_ref[...] = y * dinv

  return pl.pallas_call(
      body,
      grid=(2 * (N // BLK),),
      in_specs=[
          pl.BlockSpec((BLK, 256), lambda i: (i % (N // BLK), 0)),
          pl.BlockSpec((256, HH), lambda i: (0, i // (N // BLK))),
          pl.BlockSpec((BLK, 128), lambda i: (i % (N // BLK), 0)),
          pl.BlockSpec((BLK, 128), lambda i: (i % (N // BLK), 0)),
      ],
      out_specs=pl.BlockSpec((BLK, HH), lambda i: (i, 0)),
      out_shape=jax.ShapeDtypeStruct((2 * N, HH), jnp.float32),
  )(x, W, hist0, hist1)


def _tc_layer(a0, a1, ys_all, hist0, hist1, b_prev, W):
  def body(a0_r, a1_r, p0_r, p1_r, h0_r, h1_r, b_r, w_r, o_ref):
    dinv = _dinv_of(h0_r[...], h1_r[...])
    a = jnp.concatenate([a0_r[...], a1_r[...]], axis=1)
    p = jnp.concatenate([p0_r[...], p1_r[...]], axis=1)
    h = jnp.maximum(dinv * (a + p) + b_r[...], 0.0)
    y = jnp.dot(h, w_r[...], preferred_element_type=jnp.float32,
                precision=lax.Precision.HIGHEST)
    o_ref[...] = y * dinv

  nb = N // BLK
  return pl.pallas_call(
      body,
      grid=(2 * nb,),
      in_specs=[
          pl.BlockSpec((BLK, HH), lambda i: (i % nb, 0)),
          pl.BlockSpec((BLK, HH), lambda i: (i % nb, 0)),
          pl.BlockSpec((BLK, HH), lambda i: (i % nb, 0)),
          pl.BlockSpec((BLK, HH), lambda i: (nb + i % nb, 0)),
          pl.BlockSpec((BLK, 128), lambda i: (i % nb, 0)),
          pl.BlockSpec((BLK, 128), lambda i: (i % nb, 0)),
          pl.BlockSpec((1, 256), lambda i: (0, 0)),
          pl.BlockSpec((256, HH), lambda i: (0, i // nb)),
      ],
      out_specs=pl.BlockSpec((BLK, HH), lambda i: (i, 0)),
      out_shape=jax.ShapeDtypeStruct((2 * N, HH), jnp.float32),
  )(a0, a1, ys_all, ys_all, hist0, hist1, b_prev, W)


def _tc_pool(a0, a1, ys_all, hist0, hist1, b3, batch2d):
  def body(a0_r, a1_r, p0_r, p1_r, h0_r, h1_r, b_r, bat_r, ps_ref, cnt_ref):
    i = pl.program_id(0)
    dinv = _dinv_of(h0_r[...], h1_r[...])
    a = jnp.concatenate([a0_r[...], a1_r[...]], axis=1)
    p = jnp.concatenate([p0_r[...], p1_r[...]], axis=1)
    mol = dinv * (a + p) + b_r[...]
    gids = lax.broadcasted_iota(jnp.int32, (G, BLK), 0)
    S = (bat_r[...][0] == gids)
    part = jnp.dot(S.astype(jnp.float32), mol,
                   preferred_element_type=jnp.float32,
                   precision=lax.Precision.HIGHEST)
    c_part = jnp.broadcast_to(
        jnp.sum(S.astype(jnp.float32), axis=1, keepdims=True), (G, 128))

    @pl.when(i == 0)
    def _():
      ps_ref[...] = part
      cnt_ref[...] = c_part

    @pl.when(i > 0)
    def _():
      ps_ref[...] += part
      cnt_ref[...] += c_part

  return pl.pallas_call(
      body,
      grid=(N // BLK,),
      in_specs=[
          pl.BlockSpec((BLK, HH), lambda i: (i, 0)),
          pl.BlockSpec((BLK, HH), lambda i: (i, 0)),
          pl.BlockSpec((BLK, HH), lambda i: (i, 0)),
          pl.BlockSpec((BLK, HH), lambda i: (N // BLK + i, 0)),
          pl.BlockSpec((BLK, 128), lambda i: (i, 0)),
          pl.BlockSpec((BLK, 128), lambda i: (i, 0)),
          pl.BlockSpec((1, 256), lambda i: (0, 0)),
          pl.BlockSpec((1, 1, BLK), lambda i: (i, 0, 0)),
      ],
      out_specs=[
          pl.BlockSpec((G, 256), lambda i: (0, 0)),
          pl.BlockSpec((G, 128), lambda i: (0, 0)),
      ],
      out_shape=[
          jax.ShapeDtypeStruct((G, 256), jnp.float32),
          jax.ShapeDtypeStruct((G, 128), jnp.float32),
      ],
  )(a0, a1, ys_all, ys_all, hist0, hist1, b3, batch2d)


def _tc_head(ps, cnt, text, mw1, mb1, mw2, mb2, tw1, tb1, tw2, tb2,
             l1w, l1b, l2w, l2b, temp11):
  def _ln(v, w, b):
    mu = jnp.mean(v, axis=-1, keepdims=True)
    var = jnp.mean(jnp.square(v - mu), axis=-1, keepdims=True)
    return (v - mu) * lax.rsqrt(var + 1e-5) * w + b

  def body(ps_r, cnt_r, t_r, mw1_r, mb1_r, mw2_r, mb2_r, tw1_r, tb1_r,
           tw2_r, tb2_r, l1w_r, l1b_r, l2w_r, l2b_r, tmp_r,
           tx_ref, xg_ref):
    cntv = jnp.maximum(cnt_r[:, 0:1], 1.0)
    pooled = ps_r[...] / cntv
    xg = jnp.maximum(
        jnp.dot(pooled, mw1_r[...], preferred_element_type=jnp.float32,
                precision=lax.Precision.HIGHEST)
        + mb1_r[...], 0.0)
    xg = jnp.dot(xg, mw2_r[...], preferred_element_type=jnp.float32,
                precision=lax.Precision.HIGHEST) + mb2_r[...]
    tx = jnp.tanh(
        jnp.dot(t_r[...], tw1_r[...], preferred_element_type=jnp.float32,
                precision=lax.Precision.HIGHEST)
        + tb1_r[...])
    tx = jnp.dot(tx, tw2_r[...], preferred_element_type=jnp.float32,
                precision=lax.Precision.HIGHEST) + tb2_r[...]
    sc = jnp.exp(tmp_r[0, 0])
    xg_ref[...] = _ln(xg, l1w_r[...], l1b_r[...]) * sc
    tx_ref[...] = _ln(tx, l2w_r[...], l2b_r[...]) * sc

  return pl.pallas_call(
      body,
      out_shape=[
          jax.ShapeDtypeStruct((G, 256), jnp.float32),
          jax.ShapeDtypeStruct((G, 256), jnp.float32),
      ],
  )(ps, cnt, text, mw1, mb1, mw2, mb2, tw1, tb1, tw2, tb2,
    l1w, l1b, l2w, l2b, temp11)


def kernel(x, edge_index, batch, ptr, text_output, conv1_W, conv1_b,
           conv2_W, conv2_b, conv3_W, conv3_b, mol_h1_W, mol_h1_b,
           mol_h2_W, mol_h2_b, text_h1_W, text_h1_b, text_h2_W, text_h2_b,
           ln1_w, ln1_b, ln2_w, ln2_b, temp):
  src_both = jnp.concatenate(
      [edge_index[0], edge_index[0] + N]).reshape(2 * NBATCH, 1, K)
  dst3d = edge_index[1].reshape(NBATCH, 1, K)
  zeros_c = jnp.zeros((K, 128), jnp.float32)
  ones_c = jnp.ones((K, 128), jnp.float32)

  hist = _sc_hist(dst3d, zeros_c, ones_c)
  hist0, hist1 = hist[:N_PAD], hist[N_PAD:]
  ys = _tc_layer1(x, conv1_W, hist0, hist1)
  agg = _sc_agg(ys, src_both, dst3d, zeros_c)
  a0, a1 = agg[:N_PAD], agg[N_PAD:]
  ys = _tc_layer(a0, a1, ys, hist0, hist1, conv1_b.reshape(1, 256), conv2_W)
  agg = _sc_agg(ys, src_both, dst3d, zeros_c)
  a0, a1 = agg[:N_PAD], agg[N_PAD:]
  ys = _tc_layer(a0, a1, ys, hist0, hist1, conv2_b.reshape(1, 256), conv3_W)
  agg = _sc_agg(ys, src_both, dst3d, zeros_c)
  a0, a1 = agg[:N_PAD], agg[N_PAD:]
  ps, cnt = _tc_pool(a0, a1, ys, hist0, hist1,
                     conv3_b.reshape(1, 256), batch.reshape(N // BLK, 1, BLK))
  tx, xg = _tc_head(
      ps, cnt, text_output[0], mol_h1_W, mol_h1_b.reshape(1, 512),
      mol_h2_W, mol_h2_b.reshape(1, 256), text_h1_W, text_h1_b.reshape(1, 512),
      text_h2_W, text_h2_b.reshape(1, 256), ln1_w.reshape(1, 256),
      ln1_b.reshape(1, 256), ln2_w.reshape(1, 256), ln2_b.reshape(1, 256),
      temp.reshape(1, 1))
  return (tx, xg)
